# Initial kernel scaffold; baseline (speedup 1.0000x reference)
import jax
import jax.numpy as jnp
from jax.experimental import pallas as pl


def kernel(pos):
    raise NotImplementedError("write your pallas kernel here")


# speedup vs baseline: 1.2144x; 1.2144x over previous
import jax
import jax.numpy as jnp
from jax.experimental import pallas as pl

B = 8
N = 2048
K = 16
TR = 256

_BIG = 3.0e38
_IBIG = 1 << 30


def _knn_body(xt_ref, xT_ref, idx_ref):
    b = pl.program_id(0)
    xt = xt_ref[0]
    xT = xT_ref[0]

    cross = jax.lax.dot_general(xt, xT, (((1,), (0,)), ((), ())),
                                preferred_element_type=jnp.float32)
    xt2 = jnp.sum(xt * xt, axis=1, keepdims=True)
    xb2 = jnp.sum(xT * xT, axis=0, keepdims=True)
    d2 = jnp.maximum(xt2 + xb2 - 2.0 * cross, 0.0)

    lane = jax.lax.broadcasted_iota(jnp.int32, (TR, N), 1)
    kcol = jax.lax.broadcasted_iota(jnp.int32, (TR, K), 1)

    idx_acc = jnp.zeros((TR, K), jnp.int32)
    for t in range(K):
        m = jnp.min(d2, axis=1, keepdims=True)
        isel = jnp.min(jnp.where(d2 == m, lane, _IBIG), axis=1,
                       keepdims=True)
        d2 = jnp.where(lane == isel, _BIG, d2)
        idx_acc = jnp.where(kcol == t, isel, idx_acc)

    idx_ref[0] = idx_acc + b * N


def _knn_pallas(x, interpret=False):
    xT = jnp.transpose(x, (0, 2, 1))
    return pl.pallas_call(
        _knn_body,
        grid=(B, N // TR),
        in_specs=[
            pl.BlockSpec((1, TR, 3), lambda b, r: (b, r, 0)),
            pl.BlockSpec((1, 3, N), lambda b, r: (b, 0, 0)),
        ],
        out_specs=pl.BlockSpec((1, TR, K), lambda b, r: (b, r, 0)),
        out_shape=jax.ShapeDtypeStruct((B, N, K), jnp.int32),
        interpret=interpret,
    )(x, xT)


def kernel(pos):
    pos = pos.astype(jnp.float32)
    sid_euc = _knn_pallas(pos).reshape(-1)
    tid = jnp.repeat(jnp.arange(B * N, dtype=jnp.int32), K)

    pos_ = pos.reshape(B * N, -1)
    euc_diff = pos_[sid_euc] - pos_[tid]
    ed = euc_diff.reshape(B, N, K, -1)
    cov = jnp.einsum('bnkd,bnke->bnde', ed, ed)
    eig, vec = jnp.linalg.eigh(cov)

    sid_eig = _knn_pallas(eig).reshape(-1)

    return ((sid_euc, tid), (sid_eig, tid), (eig, vec))


# speedup vs baseline: 18.8750x; 15.5421x over previous
import jax
import jax.numpy as jnp
from jax.experimental import pallas as pl

B = 8
N = 2048
K = 16
TR = 256

M = B * N
ER = 8
EG = M // (ER * 128)

_BIG = 3.0e38
_IBIG = 1 << 30


def _knn_body(xt_ref, xT_ref, idx_ref):
    b = pl.program_id(0)
    xt = xt_ref[0]
    xT = xT_ref[0]

    cross = jax.lax.dot_general(xt, xT, (((1,), (0,)), ((), ())),
                                preferred_element_type=jnp.float32)
    xt2 = jnp.sum(xt * xt, axis=1, keepdims=True)
    xb2 = jnp.sum(xT * xT, axis=0, keepdims=True)
    d2 = jnp.maximum(xt2 + xb2 - 2.0 * cross, 0.0)

    lane = jax.lax.broadcasted_iota(jnp.int32, (TR, N), 1)
    kcol = jax.lax.broadcasted_iota(jnp.int32, (TR, K), 1)

    idx_acc = jnp.zeros((TR, K), jnp.int32)
    for t in range(K):
        m = jnp.min(d2, axis=1, keepdims=True)
        isel = jnp.min(jnp.where(d2 == m, lane, _IBIG), axis=1,
                       keepdims=True)
        d2 = jnp.where(lane == isel, _BIG, d2)
        idx_acc = jnp.where(kcol == t, isel, idx_acc)

    idx_ref[0] = idx_acc + b * N


def _knn_pallas(x, interpret=False):
    xT = jnp.transpose(x, (0, 2, 1))
    return pl.pallas_call(
        _knn_body,
        grid=(B, N // TR),
        in_specs=[
            pl.BlockSpec((1, TR, 3), lambda b, r: (b, r, 0)),
            pl.BlockSpec((1, 3, N), lambda b, r: (b, 0, 0)),
        ],
        out_specs=pl.BlockSpec((1, TR, K), lambda b, r: (b, r, 0)),
        out_shape=jax.ShapeDtypeStruct((B, N, K), jnp.int32),
        interpret=interpret,
    )(x, xT)


def _eigh_body(c_ref, w_ref, v_ref):
    one = jnp.float32(1.0)
    zero = jnp.float32(0.0)

    a = [[c_ref[3 * i + j] for j in range(3)] for i in range(3)]
    fr2 = sum(a[i][j] * a[i][j] for i in range(3) for j in range(3))
    thresh = jnp.float32(1e-5) * jnp.sqrt(fr2)

    ones = jnp.full_like(a[0][0], one)
    zeros = jnp.zeros_like(a[0][0])
    V = [[ones if i == j else zeros for j in range(3)] for i in range(3)]

    for _ in range(8):
        off = jnp.sqrt(2.0 * (a[0][1] * a[0][1] + a[0][2] * a[0][2]
                              + a[1][2] * a[1][2]))
        conv = off <= thresh
        for (p, q) in ((0, 2), (1, 2), (0, 1)):
            y = a[p][q]
            skip = conv | (y == zero)
            tau = (a[q][q] - a[p][p]) / (2.0 * y)
            sg = jnp.where(tau >= zero, one, -one)
            t = sg / (jnp.abs(tau) + jnp.sqrt(one + tau * tau))
            c = one / jnp.sqrt(one + t * t)
            s = t * c
            c = jnp.where(skip, one, c)
            s = jnp.where(skip, zero, s)
            ap = [row[:] for row in a]
            for j in range(3):
                ap[p][j] = c * a[p][j] - s * a[q][j]
                ap[q][j] = s * a[p][j] + c * a[q][j]
            a = [row[:] for row in ap]
            for i in range(3):
                a[i][p] = c * ap[i][p] - s * ap[i][q]
                a[i][q] = s * ap[i][p] + c * ap[i][q]
            vp = [row[:] for row in V]
            for i in range(3):
                vp[i][p] = c * V[i][p] - s * V[i][q]
                vp[i][q] = s * V[i][p] + c * V[i][q]
            V = vp

    w = [a[0][0], a[1][1], a[2][2]]
    for (i, j) in ((0, 1), (1, 2), (0, 1)):
        cond = w[j] < w[i]
        w[i], w[j] = (jnp.where(cond, w[j], w[i]),
                      jnp.where(cond, w[i], w[j]))
        for r in range(3):
            V[r][i], V[r][j] = (jnp.where(cond, V[r][j], V[r][i]),
                                jnp.where(cond, V[r][i], V[r][j]))

    for k in range(3):
        w_ref[k] = w[k]
    for i in range(3):
        for j in range(3):
            v_ref[3 * i + j] = V[i][j]


def _eigh_pallas(cov, interpret=False):
    planes = cov.reshape(M, 9).T.reshape(9, M // 128, 128)
    w, v = pl.pallas_call(
        _eigh_body,
        grid=(EG,),
        in_specs=[pl.BlockSpec((9, ER, 128), lambda i: (0, i, 0))],
        out_specs=[
            pl.BlockSpec((3, ER, 128), lambda i: (0, i, 0)),
            pl.BlockSpec((9, ER, 128), lambda i: (0, i, 0)),
        ],
        out_shape=[
            jax.ShapeDtypeStruct((3, M // 128, 128), jnp.float32),
            jax.ShapeDtypeStruct((9, M // 128, 128), jnp.float32),
        ],
        interpret=interpret,
    )(planes)
    eig = w.reshape(3, M).T
    vec = v.reshape(3, 3, M).transpose(2, 0, 1)
    return eig, vec


def kernel(pos):
    pos = pos.astype(jnp.float32)
    sid_euc = _knn_pallas(pos).reshape(-1)
    tid = jnp.repeat(jnp.arange(B * N, dtype=jnp.int32), K)

    pos_ = pos.reshape(B * N, -1)
    euc_diff = pos_[sid_euc] - pos_[tid]
    ed = euc_diff.reshape(B, N, K, -1)
    cov = jnp.einsum('bnkd,bnke->bnde', ed, ed)
    eig, vec = _eigh_pallas(cov.reshape(M, 3, 3))
    eig = eig.reshape(B, N, 3)
    vec = vec.reshape(B, N, 3, 3)

    sid_eig = _knn_pallas(eig).reshape(-1)

    return ((sid_euc, tid), (sid_eig, tid), (eig, vec))
